# baseline (speedup 1.0000x reference)
import functools

import jax
import jax.numpy as jnp
from jax import lax
from jax.experimental import pallas as pl
from jax.experimental.pallas import tpu as pltpu
from jax.experimental.pallas import tpu_sc as plsc

B = 16384
F = 26
V = 1000000

NC = 2
NS = 16
NW = NC * NS
RPW = B // NW
CHW = 128
CH = RPW // CHW
L = 16
KH = 10


def _build():
    mesh = plsc.VectorSubcoreMesh(core_axis_name="c", subcore_axis_name="s")

    @functools.partial(
        pl.kernel,
        mesh=mesh,
        out_type=jax.ShapeDtypeStruct((NW, CH, CHW), jnp.float32),
        scratch_types=[
            pltpu.VMEM((F * RPW,), jnp.int32),
            pltpu.VMEM((F * RPW,), jnp.float32),
            pltpu.VMEM((CH, CHW), jnp.float32),
            pltpu.VMEM_SHARED((1, V), jnp.float32),
            pltpu.SemaphoreType.DMA,
            pltpu.SemaphoreType.DMA,
            pltpu.SemaphoreType.DMA((F,)),
        ],
    )
    def k(xt_hbm, tab_hbm, out_hbm, idx_v, col_v, res_v, spm, sem_s, sem_t, sems):
        cid = lax.axis_index("c")
        sid = lax.axis_index("s")
        wid = sid * NC + cid

        @pl.when(sid == 0)
        def _():
            pltpu.async_copy(tab_hbm, spm, sem_t)

        def stage(f, carry):
            pltpu.async_copy(
                xt_hbm.at[f, pl.ds(wid * RPW, RPW)],
                idx_v.at[pl.ds(f * RPW, RPW)],
                sem_s,
            )
            return carry

        lax.fori_loop(0, F, stage, 0)
        for f in range(F):
            pltpu.make_async_copy(
                xt_hbm.at[f, pl.ds(wid * RPW, RPW)],
                idx_v.at[pl.ds(f * RPW, RPW)],
                sem_s,
            ).wait()

        def fire_hbm(f, carry):
            pltpu.async_copy(
                tab_hbm.at[0].at[idx_v.at[pl.ds(f * RPW, RPW)]],
                col_v.at[pl.ds(f * RPW, RPW)],
                sems.at[f],
            )
            return carry

        lax.fori_loop(0, KH, fire_hbm, 0)

        @pl.when(sid == 0)
        def _():
            pltpu.make_async_copy(tab_hbm, spm, sem_t).wait()
        plsc.subcore_barrier()

        def fire(f, carry):
            pltpu.async_copy(
                spm.at[0].at[idx_v.at[pl.ds(f * RPW, RPW)]],
                col_v.at[pl.ds(f * RPW, RPW)],
                sems.at[f],
            )
            return carry

        lax.fori_loop(KH, F, fire, 0)

        acc = [jnp.zeros((L,), jnp.float32) for _ in range(CH * (CHW // L))]
        for f in range(F):
            src = tab_hbm.at[0] if f < KH else spm.at[0]
            pltpu.make_async_copy(
                src.at[idx_v.at[pl.ds(f * RPW, RPW)]],
                col_v.at[pl.ds(f * RPW, RPW)],
                sems.at[f],
            ).wait()
            for q in range(RPW // L):
                acc[q] = acc[q] + col_v[pl.ds(f * RPW + q * L, L)]

        for j in range(CH):
            for t in range(CHW // L):
                res_v[j, pl.ds(t * L, L)] = acc[j * (CHW // L) + t]
        pltpu.sync_copy(res_v, out_hbm.at[wid])

    return k


_kernel = _build()


def kernel(X, table):
    out = _kernel(X.T, table.reshape(1, V))
    return out.reshape(B, 1)


# speedup vs baseline: 1.0338x; 1.0338x over previous
import functools

import jax
import jax.numpy as jnp
from jax import lax
from jax.experimental import pallas as pl
from jax.experimental.pallas import tpu as pltpu
from jax.experimental.pallas import tpu_sc as plsc

B = 16384
F = 26
V = 1000000

NC = 2
NS = 16
NW = NC * NS
RPW = B // NW
CHW = 128
CH = RPW // CHW
L = 16
KH = 4


def _build():
    mesh = plsc.VectorSubcoreMesh(core_axis_name="c", subcore_axis_name="s")

    @functools.partial(
        pl.kernel,
        mesh=mesh,
        out_type=jax.ShapeDtypeStruct((NW, CH, CHW), jnp.float32),
        scratch_types=[
            pltpu.VMEM((F * RPW,), jnp.int32),
            pltpu.VMEM((F * RPW,), jnp.float32),
            pltpu.VMEM((CH, CHW), jnp.float32),
            pltpu.VMEM_SHARED((1, V), jnp.float32),
            pltpu.SemaphoreType.DMA,
            pltpu.SemaphoreType.DMA,
            pltpu.SemaphoreType.DMA((F,)),
        ],
    )
    def k(xt_hbm, tab_hbm, out_hbm, idx_v, col_v, res_v, spm, sem_s, sem_t, sems):
        cid = lax.axis_index("c")
        sid = lax.axis_index("s")
        wid = sid * NC + cid

        @pl.when(sid == 0)
        def _():
            pltpu.async_copy(tab_hbm, spm, sem_t)

        def stage(f, carry):
            pltpu.async_copy(
                xt_hbm.at[f, pl.ds(wid * RPW, RPW)],
                idx_v.at[pl.ds(f * RPW, RPW)],
                sem_s,
            )
            return carry

        lax.fori_loop(0, F, stage, 0)
        for f in range(F):
            pltpu.make_async_copy(
                xt_hbm.at[f, pl.ds(wid * RPW, RPW)],
                idx_v.at[pl.ds(f * RPW, RPW)],
                sem_s,
            ).wait()

        def fire_hbm(f, carry):
            pltpu.async_copy(
                tab_hbm.at[0].at[idx_v.at[pl.ds(f * RPW, RPW)]],
                col_v.at[pl.ds(f * RPW, RPW)],
                sems.at[f],
            )
            return carry

        lax.fori_loop(0, KH, fire_hbm, 0)

        @pl.when(sid == 0)
        def _():
            pltpu.make_async_copy(tab_hbm, spm, sem_t).wait()
        plsc.subcore_barrier()

        def fire(f, carry):
            pltpu.async_copy(
                spm.at[0].at[idx_v.at[pl.ds(f * RPW, RPW)]],
                col_v.at[pl.ds(f * RPW, RPW)],
                sems.at[f],
            )
            return carry

        lax.fori_loop(KH, F, fire, 0)

        acc = [jnp.zeros((L,), jnp.float32) for _ in range(CH * (CHW // L))]
        for f in range(F):
            src = tab_hbm.at[0] if f < KH else spm.at[0]
            pltpu.make_async_copy(
                src.at[idx_v.at[pl.ds(f * RPW, RPW)]],
                col_v.at[pl.ds(f * RPW, RPW)],
                sems.at[f],
            ).wait()
            for q in range(RPW // L):
                acc[q] = acc[q] + col_v[pl.ds(f * RPW + q * L, L)]

        for j in range(CH):
            for t in range(CHW // L):
                res_v[j, pl.ds(t * L, L)] = acc[j * (CHW // L) + t]
        pltpu.sync_copy(res_v, out_hbm.at[wid])

    return k


_kernel = _build()


def kernel(X, table):
    out = _kernel(X.T, table.reshape(1, V))
    return out.reshape(B, 1)


# speedup vs baseline: 1.0473x; 1.0130x over previous
import functools

import jax
import jax.numpy as jnp
from jax import lax
from jax.experimental import pallas as pl
from jax.experimental.pallas import tpu as pltpu
from jax.experimental.pallas import tpu_sc as plsc

B = 16384
F = 26
V = 1000000

NC = 2
NS = 16
NW = NC * NS
RPW = B // NW
CHW = 128
CH = RPW // CHW
L = 16
KH = 6


def _build():
    mesh = plsc.VectorSubcoreMesh(core_axis_name="c", subcore_axis_name="s")

    @functools.partial(
        pl.kernel,
        mesh=mesh,
        out_type=jax.ShapeDtypeStruct((NW, CH, CHW), jnp.float32),
        scratch_types=[
            pltpu.VMEM((F * RPW,), jnp.int32),
            pltpu.VMEM((F * RPW,), jnp.float32),
            pltpu.VMEM((CH, CHW), jnp.float32),
            pltpu.VMEM_SHARED((1, V), jnp.float32),
            pltpu.SemaphoreType.DMA,
            pltpu.SemaphoreType.DMA,
            pltpu.SemaphoreType.DMA((F,)),
        ],
    )
    def k(xt_hbm, tab_hbm, out_hbm, idx_v, col_v, res_v, spm, sem_s, sem_t, sems):
        cid = lax.axis_index("c")
        sid = lax.axis_index("s")
        wid = sid * NC + cid

        @pl.when(sid == 0)
        def _():
            pltpu.async_copy(tab_hbm, spm, sem_t)

        def stage(f, carry):
            pltpu.async_copy(
                xt_hbm.at[f, pl.ds(wid * RPW, RPW)],
                idx_v.at[pl.ds(f * RPW, RPW)],
                sem_s,
            )
            return carry

        lax.fori_loop(0, F, stage, 0)
        for f in range(F):
            pltpu.make_async_copy(
                xt_hbm.at[f, pl.ds(wid * RPW, RPW)],
                idx_v.at[pl.ds(f * RPW, RPW)],
                sem_s,
            ).wait()

        def fire_hbm(f, carry):
            pltpu.async_copy(
                tab_hbm.at[0].at[idx_v.at[pl.ds(f * RPW, RPW)]],
                col_v.at[pl.ds(f * RPW, RPW)],
                sems.at[f],
            )
            return carry

        lax.fori_loop(0, KH, fire_hbm, 0)

        @pl.when(sid == 0)
        def _():
            pltpu.make_async_copy(tab_hbm, spm, sem_t).wait()
        plsc.subcore_barrier()

        def fire(f, carry):
            pltpu.async_copy(
                spm.at[0].at[idx_v.at[pl.ds(f * RPW, RPW)]],
                col_v.at[pl.ds(f * RPW, RPW)],
                sems.at[f],
            )
            return carry

        lax.fori_loop(KH, F, fire, 0)

        acc = [jnp.zeros((L,), jnp.float32) for _ in range(CH * (CHW // L))]
        for f in range(F):
            src = tab_hbm.at[0] if f < KH else spm.at[0]
            pltpu.make_async_copy(
                src.at[idx_v.at[pl.ds(f * RPW, RPW)]],
                col_v.at[pl.ds(f * RPW, RPW)],
                sems.at[f],
            ).wait()
            for q in range(RPW // L):
                acc[q] = acc[q] + col_v[pl.ds(f * RPW + q * L, L)]

        for j in range(CH):
            for t in range(CHW // L):
                res_v[j, pl.ds(t * L, L)] = acc[j * (CHW // L) + t]
        pltpu.sync_copy(res_v, out_hbm.at[wid])

    return k


_kernel = _build()


def kernel(X, table):
    out = _kernel(X.T, table.reshape(1, V))
    return out.reshape(B, 1)
